# baseline (speedup 1.0000x reference)
import functools

import jax
import jax.numpy as jnp
from jax import lax
from jax.experimental import pallas as pl
from jax.experimental.pallas import tpu as pltpu
from jax.experimental.pallas import tpu_sc as plsc

EMB_DIM = 64
NUM_CORES = 2
NUM_SUBCORES = 16
NUM_WORKERS = NUM_CORES * NUM_SUBCORES
CHUNK = 128


@functools.lru_cache(maxsize=None)
def _make_gather_kernel(flat_b: int):
    b_per_w = flat_b // NUM_WORKERS
    n_chunks = b_per_w // CHUNK
    mesh = plsc.VectorSubcoreMesh(core_axis_name="c", subcore_axis_name="s")

    @functools.partial(
        pl.kernel,
        out_type=jax.ShapeDtypeStruct((flat_b, EMB_DIM), jnp.float32),
        mesh=mesh,
        compiler_params=pltpu.CompilerParams(use_tc_tiling_on_sc=False),
        scratch_types=[
            pltpu.VMEM((n_chunks, CHUNK), jnp.int32),
            pltpu.VMEM((CHUNK, EMB_DIM), jnp.float32),
            pltpu.SemaphoreType.DMA,
        ],
    )
    def emb_kernel(idx_hbm, table_hbm, out_hbm, idx_v, rows_v, sem):
        wid = lax.axis_index("s") * NUM_CORES + lax.axis_index("c")
        base = wid * b_per_w
        pltpu.sync_copy(idx_hbm.at[wid], idx_v)

        def body(j, carry):
            pltpu.async_copy(table_hbm.at[idx_v.at[j]], rows_v, sem).wait()
            pltpu.sync_copy(rows_v, out_hbm.at[pl.ds(base + j * CHUNK, CHUNK)])
            return carry

        lax.fori_loop(0, n_chunks, body, 0)

    return emb_kernel


def kernel(input, weight):
    flat = input.reshape(-1)
    flat_b = flat.shape[0]
    idx3 = flat.reshape(NUM_WORKERS, flat_b // NUM_WORKERS // CHUNK, CHUNK)
    out = _make_gather_kernel(flat_b)(idx3, weight)
    return out.reshape(input.shape + (EMB_DIM,))


# speedup vs baseline: 1.0730x; 1.0730x over previous
import functools

import jax
import jax.numpy as jnp
from jax import lax
from jax.experimental import pallas as pl
from jax.experimental.pallas import tpu as pltpu
from jax.experimental.pallas import tpu_sc as plsc

EMB_DIM = 64
NUM_CORES = 2
NUM_SUBCORES = 16
NUM_WORKERS = NUM_CORES * NUM_SUBCORES
CHUNK = 128


@functools.lru_cache(maxsize=None)
def _make_gather_kernel(flat_b: int):
    b_per_w = flat_b // NUM_WORKERS
    n_chunks = b_per_w // CHUNK
    mesh = plsc.VectorSubcoreMesh(core_axis_name="c", subcore_axis_name="s")

    nbuf = 8
    n_groups = n_chunks // nbuf

    @functools.partial(
        pl.kernel,
        out_type=jax.ShapeDtypeStruct((flat_b, EMB_DIM), jnp.float32),
        mesh=mesh,
        compiler_params=pltpu.CompilerParams(use_tc_tiling_on_sc=False),
        scratch_types=[
            pltpu.VMEM((n_chunks, CHUNK), jnp.int32),
            pltpu.VMEM((nbuf, CHUNK, EMB_DIM), jnp.float32),
            pltpu.SemaphoreType.DMA((nbuf,)),
            pltpu.SemaphoreType.DMA((nbuf,)),
        ],
    )
    def emb_kernel(idx_hbm, table_hbm, out_hbm, idx_v, rows_v, gsem, wsem):
        wid = lax.axis_index("s") * NUM_CORES + lax.axis_index("c")
        base = wid * b_per_w
        pltpu.sync_copy(idx_hbm.at[wid], idx_v)

        def gather(j, b):
            return pltpu.async_copy(
                table_hbm.at[idx_v.at[j]], rows_v.at[b], gsem.at[b])

        def writeback(j, b):
            return pltpu.async_copy(
                rows_v.at[b], out_hbm.at[pl.ds(base + j * CHUNK, CHUNK)],
                wsem.at[b])

        def wb_drain(b):
            pltpu.make_async_copy(
                rows_v.at[b], out_hbm.at[pl.ds(base, CHUNK)], wsem.at[b]
            ).wait()

        gathers = [gather(b, b) for b in range(nbuf)]
        wbs = []
        for b in range(nbuf):
            gathers[b].wait()
            wbs.append(writeback(b, b))

        def group(p, carry):
            j0 = p * nbuf
            gs = []
            for b in range(nbuf):
                wb_drain(b)
                gs.append(gather(j0 + b, b))
            for b in range(nbuf):
                gs[b].wait()
                writeback(j0 + b, b)
            return carry

        lax.fori_loop(1, n_groups, group, 0)

        for b in range(nbuf):
            wb_drain(b)

    return emb_kernel


def kernel(input, weight):
    flat = input.reshape(-1)
    flat_b = flat.shape[0]
    idx3 = flat.reshape(NUM_WORKERS, flat_b // NUM_WORKERS // CHUNK, CHUNK)
    out = _make_gather_kernel(flat_b)(idx3, weight)
    return out.reshape(input.shape + (EMB_DIM,))
